# baseline (speedup 1.0000x reference)
import jax
import jax.numpy as jnp
from jax import lax
from jax.experimental import pallas as pl
from jax.experimental.pallas import tpu as pltpu
from jax.experimental.pallas import tpu_sc as plsc

NUM_EMB = 1_000_000
DIM = 64
OUT_DIM = 128
BATCH = 16384
PADDED = 128

NC, NS = 2, 16
NW = NC * NS
B_PER_W = BATCH // NW
NSLOT = 5


def _gather_kernel(idx_hbm, tableT_hbm, out_hbm, idx_v, idx_s, chunks, rows_v, sem):
    wid = lax.axis_index("s") * NC + lax.axis_index("c")
    base = wid * B_PER_W
    pltpu.sync_copy(idx_hbm.at[pl.ds(base, B_PER_W)], idx_v)

    def stage(s, carry):
        iv = idx_v[pl.ds(16 * s, 16)]
        for t in range(16):
            idx_s[16 * s + t] = iv[t]
        return carry

    lax.fori_loop(0, B_PER_W // 16, stage, 0)

    def blk_copy(k):
        c = pl.multiple_of((idx_s[k] >> 7) << 7, 128)
        return pltpu.make_async_copy(
            tableT_hbm.at[:, pl.ds(c, 128)],
            chunks.at[k % NSLOT],
            sem,
        )

    def extract(k):
        blk_copy(k).wait()
        l = idx_s[k] & 127
        lvec = jnp.full((16,), l, jnp.int32)
        for m in range(DIM // 16):
            jvec = lax.iota(jnp.int32, 16) + (16 * m)
            vals = plsc.load_gather(chunks.at[k % NSLOT], [jvec, lvec])
            rows_v[k, pl.ds(16 * m, 16)] = vals

    def prime(k, carry):
        blk_copy(k).start()
        return carry

    def body(k, carry):
        extract(k - NSLOT)
        blk_copy(k).start()
        return carry

    def tail(k, carry):
        extract(k)
        return carry

    lax.fori_loop(0, NSLOT, prime, 0)
    lax.fori_loop(NSLOT, B_PER_W, body, 0)
    lax.fori_loop(B_PER_W - NSLOT, B_PER_W, tail, 0)
    pltpu.sync_copy(rows_v, out_hbm.at[pl.ds(base, B_PER_W)])


@jax.jit
def _sc_gather(idx, tableT):
    mesh = plsc.VectorSubcoreMesh(core_axis_name="c", subcore_axis_name="s")
    return pl.kernel(
        _gather_kernel,
        out_type=jax.ShapeDtypeStruct((BATCH, PADDED), jnp.float32),
        mesh=mesh,
        scratch_types=[
            pltpu.VMEM((B_PER_W,), jnp.int32),
            pltpu.SMEM((B_PER_W,), jnp.int32),
            pltpu.VMEM((NSLOT, DIM, 128), jnp.float32),
            pltpu.VMEM((B_PER_W, PADDED), jnp.float32),
            pltpu.SemaphoreType.DMA,
        ],
        compiler_params=pltpu.CompilerParams(
            use_tc_tiling_on_sc=True, needs_layout_passes=False
        ),
    )(idx, tableT)


BLK = 2048


def _mm_kernel(g_ref, wt_ref, b_ref, o_ref):
    h = jnp.maximum(g_ref[:, :DIM], 0.0)
    o_ref[...] = (
        jnp.dot(h, wt_ref[...], preferred_element_type=jnp.float32) + b_ref[...]
    )


@jax.jit
def _tc_head(g, wt, b2d):
    return pl.pallas_call(
        _mm_kernel,
        grid=(BATCH // BLK,),
        in_specs=[
            pl.BlockSpec((BLK, PADDED), lambda i: (i, 0)),
            pl.BlockSpec((DIM, OUT_DIM), lambda i: (0, 0)),
            pl.BlockSpec((1, OUT_DIM), lambda i: (0, 0)),
        ],
        out_specs=pl.BlockSpec((BLK, OUT_DIM), lambda i: (i, 0)),
        out_shape=jax.ShapeDtypeStruct((BATCH, OUT_DIM), jnp.float32),
    )(g, wt, b2d)


def kernel(env_index, table, W, b):
    idx = env_index.astype(jnp.int32)
    g = _sc_gather(idx, table.T)
    return _tc_head(g, W.T, b.reshape(1, OUT_DIM))


# speedup vs baseline: 1.5045x; 1.5045x over previous
import jax
import jax.numpy as jnp
from jax import lax
from jax.experimental import pallas as pl
from jax.experimental.pallas import tpu as pltpu
from jax.experimental.pallas import tpu_sc as plsc

NUM_EMB = 1_000_000
DIM = 64
OUT_DIM = 128
BATCH = 16384
PADDED = 128

NC, NS = 2, 16
NW = NC * NS
B_PER_W = BATCH // NW
NSLOT = 6
QROWS = 128


def _gather_kernel(sidx_hbm, ord_hbm, tableT_hbm, out_hbm,
                   idx_v, idx_s, ord_v, chunks, rows_v, sem, osem):
    wid = lax.axis_index("s") * NC + lax.axis_index("c")
    base = wid * B_PER_W
    pltpu.sync_copy(sidx_hbm.at[pl.ds(base, B_PER_W)], idx_v)
    pltpu.sync_copy(ord_hbm.at[wid], ord_v)

    def stage(s, carry):
        iv = idx_v[pl.ds(16 * s, 16)]
        for t in range(16):
            idx_s[16 * s + t] = iv[t]
        return carry

    lax.fori_loop(0, B_PER_W // 16, stage, 0)

    def need(k):
        first = k == 0
        prev = idx_s[jnp.maximum(k - 1, 0)] >> 7
        return jnp.logical_or(first, (idx_s[k] >> 7) != prev)

    def blk_copy(k, slot):
        c = pl.multiple_of((idx_s[k] >> 7) << 7, 128)
        return pltpu.make_async_copy(
            tableT_hbm.at[:, pl.ds(c, 128)],
            chunks.at[slot],
            sem,
        )

    def fire(k, fcf):
        nd = need(k)
        fcf = fcf + nd.astype(jnp.int32)
        slot = (fcf - 1) % NSLOT

        @pl.when(nd)
        def _():
            blk_copy(k, slot).start()

        return fcf

    def extract(k, fce):
        nd = need(k)
        fce = fce + nd.astype(jnp.int32)
        slot = (fce - 1) % NSLOT

        @pl.when(nd)
        def _():
            blk_copy(k, slot).wait()

        l = idx_s[k] & 127
        lvec = jnp.full((16,), l, jnp.int32)
        for m in range(DIM // 16):
            jvec = lax.iota(jnp.int32, 16) + (16 * m)
            vals = plsc.load_gather(chunks.at[slot], [jvec, lvec])
            rows_v[k, pl.ds(16 * m, 16)] = vals
        return fce

    def prime(k, fcf):
        return fire(k, fcf)

    def body(k, carry):
        fce, fcf = carry
        fce = extract(k - NSLOT, fce)
        fcf = fire(k, fcf)
        return fce, fcf

    def tail(k, carry):
        fce, fcf = carry
        fce = extract(k, fce)
        return fce, fcf

    fcf = lax.fori_loop(0, NSLOT, prime, jnp.int32(0))
    fce, fcf = lax.fori_loop(NSLOT, B_PER_W, body, (jnp.int32(0), fcf))
    lax.fori_loop(B_PER_W - NSLOT, B_PER_W, tail, (fce, fcf))

    ocopies = []
    for q in range(B_PER_W // QROWS):
        ocopies.append(
            pltpu.make_async_copy(
                rows_v.at[pl.ds(q * QROWS, QROWS)],
                out_hbm.at[ord_v.at[q]],
                osem,
            )
        )
    for cp in ocopies:
        cp.start()
    for cp in ocopies:
        cp.wait()


@jax.jit
def _sc_gather(sidx, ord3d, tableT):
    mesh = plsc.VectorSubcoreMesh(core_axis_name="c", subcore_axis_name="s")
    return pl.kernel(
        _gather_kernel,
        out_type=jax.ShapeDtypeStruct((BATCH, PADDED), jnp.float32),
        mesh=mesh,
        scratch_types=[
            pltpu.VMEM((B_PER_W,), jnp.int32),
            pltpu.SMEM((B_PER_W,), jnp.int32),
            pltpu.VMEM((B_PER_W // QROWS, QROWS), jnp.int32),
            pltpu.VMEM((NSLOT, DIM, 128), jnp.float32),
            pltpu.VMEM((B_PER_W, PADDED), jnp.float32),
            pltpu.SemaphoreType.DMA,
            pltpu.SemaphoreType.DMA,
        ],
        compiler_params=pltpu.CompilerParams(
            use_tc_tiling_on_sc=True, needs_layout_passes=False
        ),
    )(sidx, ord3d, tableT)


BLK = 2048


def _mm_kernel(g_ref, wt_ref, b_ref, o_ref):
    h = jnp.maximum(g_ref[:, :DIM], 0.0)
    o_ref[...] = (
        jnp.dot(h, wt_ref[...], preferred_element_type=jnp.float32) + b_ref[...]
    )


@jax.jit
def _tc_head(g, wt, b2d):
    return pl.pallas_call(
        _mm_kernel,
        grid=(BATCH // BLK,),
        in_specs=[
            pl.BlockSpec((BLK, PADDED), lambda i: (i, 0)),
            pl.BlockSpec((DIM, OUT_DIM), lambda i: (0, 0)),
            pl.BlockSpec((1, OUT_DIM), lambda i: (0, 0)),
        ],
        out_specs=pl.BlockSpec((BLK, OUT_DIM), lambda i: (i, 0)),
        out_shape=jax.ShapeDtypeStruct((BATCH, OUT_DIM), jnp.float32),
    )(g, wt, b2d)


def kernel(env_index, table, W, b):
    idx = env_index.astype(jnp.int32)
    sidx, order = lax.sort_key_val(idx, lax.iota(jnp.int32, BATCH))
    ord3d = order.reshape(NW, B_PER_W // QROWS, QROWS)
    g = _sc_gather(sidx, ord3d, table.T)
    return _tc_head(g, W.T, b.reshape(1, OUT_DIM))


# speedup vs baseline: 1.8592x; 1.2358x over previous
import jax
import jax.numpy as jnp
from jax import lax
from jax.experimental import pallas as pl
from jax.experimental.pallas import tpu as pltpu
from jax.experimental.pallas import tpu_sc as plsc

NUM_EMB = 1_000_000
DIM = 64
OUT_DIM = 128
BATCH = 16384
PADDED = 128

NC, NS = 2, 16
NW = NC * NS
B_PER_W = BATCH // NW
NSLOT = 6
QROWS = 128


def _gather_kernel(sidx_hbm, ord_hbm, tableT_hbm, out_hbm,
                   idx_v, idx_s, ord_v, chunks, rows_v, sem, osem):
    wid = lax.axis_index("s") * NC + lax.axis_index("c")
    base = wid * B_PER_W
    pltpu.sync_copy(sidx_hbm.at[pl.ds(base, B_PER_W)], idx_v)
    pltpu.sync_copy(ord_hbm.at[wid], ord_v)

    def stage(s, carry):
        iv = idx_v[pl.ds(16 * s, 16)]
        for t in range(16):
            idx_s[16 * s + t] = iv[t]
        return carry

    lax.fori_loop(0, B_PER_W // 16, stage, 0)

    def need(k):
        first = k == 0
        prev = idx_s[jnp.maximum(k - 1, 0)] >> 7
        return jnp.logical_or(first, (idx_s[k] >> 7) != prev)

    def blk_copy(k, slot):
        c = pl.multiple_of((idx_s[k] >> 7) << 7, 128)
        return pltpu.make_async_copy(
            tableT_hbm.at[:, pl.ds(c, 128)],
            chunks.at[slot],
            sem,
        )

    def body(p, carry):
        kf, fcf, fce = carry

        def adv_cond(c):
            kf, fcf = c
            return jnp.logical_and(
                kf < B_PER_W,
                jnp.logical_or(kf <= p, fcf - fce < NSLOT),
            )

        def adv_body(c):
            kf, fcf = c
            nd = need(kf)

            @pl.when(nd)
            def _():
                blk_copy(kf, fcf % NSLOT).start()

            return kf + 1, fcf + nd.astype(jnp.int32)

        kf, fcf = lax.while_loop(adv_cond, adv_body, (kf, fcf))

        nd = need(p)
        fce = fce + nd.astype(jnp.int32)
        slot = (fce - 1) % NSLOT

        @pl.when(nd)
        def _():
            blk_copy(p, slot).wait()

        l = idx_s[p] & 127
        lvec = jnp.full((16,), l, jnp.int32)
        for m in range(DIM // 16):
            jvec = lax.iota(jnp.int32, 16) + (16 * m)
            vals = plsc.load_gather(chunks.at[slot], [jvec, lvec])
            rows_v[p, pl.ds(16 * m, 16)] = vals
        return kf, fcf, fce

    lax.fori_loop(
        0, B_PER_W, body, (jnp.int32(0), jnp.int32(0), jnp.int32(0))
    )

    ocopies = []
    for q in range(B_PER_W // QROWS):
        ocopies.append(
            pltpu.make_async_copy(
                rows_v.at[pl.ds(q * QROWS, QROWS)],
                out_hbm.at[ord_v.at[q]],
                osem,
            )
        )
    for cp in ocopies:
        cp.start()
    for cp in ocopies:
        cp.wait()


@jax.jit
def _sc_gather(sidx, ord3d, tableT):
    mesh = plsc.VectorSubcoreMesh(core_axis_name="c", subcore_axis_name="s")
    return pl.kernel(
        _gather_kernel,
        out_type=jax.ShapeDtypeStruct((BATCH, PADDED), jnp.float32),
        mesh=mesh,
        scratch_types=[
            pltpu.VMEM((B_PER_W,), jnp.int32),
            pltpu.SMEM((B_PER_W,), jnp.int32),
            pltpu.VMEM((B_PER_W // QROWS, QROWS), jnp.int32),
            pltpu.VMEM((NSLOT, DIM, 128), jnp.float32),
            pltpu.VMEM((B_PER_W, PADDED), jnp.float32),
            pltpu.SemaphoreType.DMA,
            pltpu.SemaphoreType.DMA,
        ],
        compiler_params=pltpu.CompilerParams(
            use_tc_tiling_on_sc=True, needs_layout_passes=False
        ),
    )(sidx, ord3d, tableT)


BLK = 2048


def _mm_kernel(g_ref, wt_ref, b_ref, o_ref):
    h = jnp.maximum(g_ref[:, :DIM], 0.0)
    o_ref[...] = (
        jnp.dot(h, wt_ref[...], preferred_element_type=jnp.float32) + b_ref[...]
    )


@jax.jit
def _tc_head(g, wt, b2d):
    return pl.pallas_call(
        _mm_kernel,
        grid=(BATCH // BLK,),
        in_specs=[
            pl.BlockSpec((BLK, PADDED), lambda i: (i, 0)),
            pl.BlockSpec((DIM, OUT_DIM), lambda i: (0, 0)),
            pl.BlockSpec((1, OUT_DIM), lambda i: (0, 0)),
        ],
        out_specs=pl.BlockSpec((BLK, OUT_DIM), lambda i: (i, 0)),
        out_shape=jax.ShapeDtypeStruct((BATCH, OUT_DIM), jnp.float32),
    )(g, wt, b2d)


def kernel(env_index, table, W, b):
    idx = env_index.astype(jnp.int32)
    sidx, order = lax.sort_key_val(idx, lax.iota(jnp.int32, BATCH))
    ord3d = order.reshape(NW, B_PER_W // QROWS, QROWS)
    g = _sc_gather(sidx, ord3d, table.T)
    return _tc_head(g, W.T, b.reshape(1, OUT_DIM))
